# baseline (speedup 1.0000x reference)
import functools

import jax
import jax.numpy as jnp
from jax import lax
from jax.experimental import pallas as pl
from jax.experimental.pallas import tpu as pltpu
from jax.experimental.pallas import tpu_sc as plsc

_NUM_CORES = 2
_NUM_SUBCORES = 16
_NUM_WORKERS = _NUM_CORES * _NUM_SUBCORES

_NB = 16


def _make_gather(num_b: int, t: int, dim: int):
  assert num_b % (_NUM_WORKERS * _NB) == 0
  slabs_per_worker = num_b // (_NUM_WORKERS * _NB)

  mesh = plsc.VectorSubcoreMesh(core_axis_name="c", subcore_axis_name="s")

  @functools.partial(
      pl.kernel,
      mesh=mesh,
      out_type=jax.ShapeDtypeStruct((num_b, t, dim), jnp.float32),
      scratch_types=[
          pltpu.VMEM((2, _NB, t), jnp.int32),
          pltpu.VMEM((2, _NB, t, dim), jnp.float32),
          pltpu.SemaphoreType.DMA,
          pltpu.SemaphoreType.DMA,
          pltpu.SemaphoreType.DMA,
          pltpu.SemaphoreType.DMA,
          pltpu.SemaphoreType.DMA,
          pltpu.SemaphoreType.DMA,
      ],
      compiler_params=pltpu.CompilerParams(use_tc_tiling_on_sc=False),
  )
  def gather_kernel(table_hbm, idx_hbm, out_hbm, idx_v, rows_v,
                    sem_i0, sem_i1, sem_g0, sem_g1, sem_w0, sem_w1):
    wid = lax.axis_index("s") * _NUM_CORES + lax.axis_index("c")
    base = wid * slabs_per_worker * _NB

    def idx_rows(g):
      return idx_hbm.at[pl.ds(base + g * _NB, _NB)]

    def out_rows(g):
      return out_hbm.at[pl.ds(base + g * _NB, _NB)]

    pltpu.async_copy(idx_rows(0), idx_v.at[0], sem_i0)
    pltpu.async_copy(idx_rows(1), idx_v.at[1], sem_i1)

    def do_slab(g, p, sem_i, sem_g, sem_w):
      idx_p = idx_v.at[p]
      rows_p = rows_v.at[p]
      pltpu.make_async_copy(idx_rows(g), idx_p, sem_i).wait()

      @pl.when(g >= 2)
      def _():
        pltpu.make_async_copy(rows_p, out_rows(g), sem_w).wait()

      copies = []
      for r in range(_NB):
        copies.append(
            pltpu.async_copy(table_hbm.at[idx_p.at[r]], rows_p.at[r], sem_g))
      for c in copies:
        c.wait()

      pltpu.async_copy(rows_p, out_rows(g), sem_w)

      @pl.when(g + 2 < slabs_per_worker)
      def _():
        pltpu.async_copy(idx_rows(g + 2), idx_p, sem_i)

    def slab_body(g, carry):
      @pl.when(g % 2 == 0)
      def _():
        do_slab(g, 0, sem_i0, sem_g0, sem_w0)

      @pl.when(g % 2 == 1)
      def _():
        do_slab(g, 1, sem_i1, sem_g1, sem_w1)

      return carry

    lax.fori_loop(0, slabs_per_worker, slab_body, 0)

    last = slabs_per_worker - 1
    pltpu.make_async_copy(rows_v.at[0], out_rows(last - 1), sem_w0).wait()
    pltpu.make_async_copy(rows_v.at[1], out_rows(last), sem_w1).wait()

  return gather_kernel


def kernel(tokens_ids, weights):
  b, t = tokens_ids.shape
  dim = weights.shape[1]
  return _make_gather(b, t, dim)(weights, tokens_ids.astype(jnp.int32))
